# Initial kernel scaffold; baseline (speedup 1.0000x reference)
import jax
import jax.numpy as jnp
from jax.experimental import pallas as pl


def kernel(vertices, faces, face_probs):
    raise NotImplementedError("write your pallas kernel here")


# speedup vs baseline: 1.1232x; 1.1232x over previous
import functools

import jax
import jax.numpy as jnp
from jax.experimental import pallas as pl
from jax.experimental.pallas import tpu as pltpu

_K = 20


def _crossing_body(nex_ref, ney_ref, nez_ref, ex_ref, ey_ref, ez_ref,
                   eax_ref, eay_ref, eaz_ref, probs_ref, out_ref):
    nex = nex_ref[...]
    ney = ney_ref[...]
    nez = nez_ref[...]
    eax = eax_ref[...]
    eay = eay_ref[...]
    eaz = eaz_ref[...]
    acc = jnp.zeros_like(nex)
    for j in range(3):
        e_x = ex_ref[:, j:j + 1]
        e_y = ey_ref[:, j:j + 1]
        e_z = ez_ref[:, j:j + 1]
        cp_x = e_y * nez - e_z * ney
        cp_y = e_z * nex - e_x * nez
        cp_z = e_x * ney - e_y * nex
        denom = (cp_x * e_x + cp_y * e_y) + cp_z * e_z
        tnum = (cp_x * nex + cp_y * ney) + cp_z * nez
        unum = (cp_x * eax + cp_y * eay) + cp_z * eaz
        t = tnum / denom
        u = unum / denom
        mask = (t >= 0.0) & (t <= 1.0) & (u >= 0.0) & (u <= 1.0)
        acc = acc + jnp.where(mask, 1.0, 0.0)
    weighted = acc * probs_ref[...]
    out_ref[...] = jnp.sum(weighted, axis=(0, 1), keepdims=True)


def kernel(vertices, faces, face_probs):
    F = faces.shape[0]
    k = min(_K, F)
    centroids = vertices[faces].mean(axis=1)
    d2 = jnp.sum((centroids[:, None, :] - centroids[None, :, :]) ** 2, axis=-1)
    _, nearest = jax.lax.top_k(-d2, k)

    perm = jnp.array([1, 2, 0])
    edges = vertices[faces[:, perm]] - vertices[faces]
    nf = faces[nearest]
    ne = vertices[nf[:, :, perm]] - vertices[nf]

    nex = ne[..., 0].reshape(F, k * 3)
    ney = ne[..., 1].reshape(F, k * 3)
    nez = ne[..., 2].reshape(F, k * 3)
    ex = edges[..., 0]
    ey = edges[..., 1]
    ez = edges[..., 2]
    eax = jnp.tile(ex, (1, k))
    eay = jnp.tile(ey, (1, k))
    eaz = jnp.tile(ez, (1, k))
    probs2 = face_probs[:F, None]

    out = pl.pallas_call(
        _crossing_body,
        out_shape=jax.ShapeDtypeStruct((1, 1), jnp.float32),
    )(nex, ney, nez, ex, ey, ez, eax, eay, eaz, probs2)
    return out[0, 0]


# speedup vs baseline: 1.3613x; 1.2119x over previous
import functools

import jax
import jax.numpy as jnp
from jax.experimental import pallas as pl
from jax.experimental.pallas import tpu as pltpu

_K = 20


def _crossing_body(nex_ref, ney_ref, nez_ref, ex_ref, ey_ref, ez_ref,
                   eax_ref, eay_ref, eaz_ref, probs_ref, out_ref):
    nex = nex_ref[...]
    ney = ney_ref[...]
    nez = nez_ref[...]
    eax = eax_ref[...]
    eay = eay_ref[...]
    eaz = eaz_ref[...]
    acc = jnp.zeros_like(nex)
    for j in range(3):
        e_x = ex_ref[:, j:j + 1]
        e_y = ey_ref[:, j:j + 1]
        e_z = ez_ref[:, j:j + 1]
        cp_x = e_y * nez - e_z * ney
        cp_y = e_z * nex - e_x * nez
        cp_z = e_x * ney - e_y * nex
        denom = (cp_x * e_x + cp_y * e_y) + cp_z * e_z
        tnum = (cp_x * nex + cp_y * ney) + cp_z * nez
        unum = (cp_x * eax + cp_y * eay) + cp_z * eaz
        t = tnum / denom
        u = unum / denom
        mask = (t >= 0.0) & (t <= 1.0) & (u >= 0.0) & (u <= 1.0)
        acc = acc + jnp.where(mask, 1.0, 0.0)
    weighted = acc * probs_ref[...]
    out_ref[...] = jnp.sum(weighted, axis=(0, 1), keepdims=True)


def _knn_body(qx_ref, qy_ref, qz_ref, cx_ref, cy_ref, cz_ref, out_ref):
    n = cx_ref.shape[1]
    dx = qx_ref[...] - cx_ref[...]
    dy = qy_ref[...] - cy_ref[...]
    dz = qz_ref[...] - cz_ref[...]
    D = (dx * dx + dy * dy) + dz * dz
    lanes = jax.lax.broadcasted_iota(jnp.int32, D.shape, 1)
    cols = []
    for _ in range(_K):
        m = jnp.min(D, axis=1, keepdims=True)
        idxc = jnp.where(D == m, lanes, n)
        imin = jnp.min(idxc, axis=1, keepdims=True)
        cols.append(imin)
        D = jnp.where(lanes == imin, jnp.inf, D)
    out_ref[...] = jnp.concatenate(cols, axis=1)


def _knn_pallas(centroids):
    F = centroids.shape[0]
    ct = centroids.T
    bq = 8
    grid = (F // bq,)
    qspec = pl.BlockSpec((bq, 1), lambda i: (i, 0))
    cspec = pl.BlockSpec((1, F), lambda i: (0, 0))
    return pl.pallas_call(
        _knn_body,
        grid=grid,
        in_specs=[qspec, qspec, qspec, cspec, cspec, cspec],
        out_specs=pl.BlockSpec((bq, _K), lambda i: (i, 0)),
        out_shape=jax.ShapeDtypeStruct((F, _K), jnp.int32),
    )(centroids[:, 0:1], centroids[:, 1:2], centroids[:, 2:3],
      ct[0:1], ct[1:2], ct[2:3])


def kernel(vertices, faces, face_probs):
    F = faces.shape[0]
    k = min(_K, F)
    centroids = vertices[faces].mean(axis=1)
    nearest = _knn_pallas(centroids)

    perm = jnp.array([1, 2, 0])
    edges = vertices[faces[:, perm]] - vertices[faces]
    nf = faces[nearest]
    ne = vertices[nf[:, :, perm]] - vertices[nf]

    nex = ne[..., 0].reshape(F, k * 3)
    ney = ne[..., 1].reshape(F, k * 3)
    nez = ne[..., 2].reshape(F, k * 3)
    ex = edges[..., 0]
    ey = edges[..., 1]
    ez = edges[..., 2]
    eax = jnp.tile(ex, (1, k))
    eay = jnp.tile(ey, (1, k))
    eaz = jnp.tile(ez, (1, k))
    probs2 = face_probs[:F, None]

    out = pl.pallas_call(
        _crossing_body,
        out_shape=jax.ShapeDtypeStruct((1, 1), jnp.float32),
    )(nex, ney, nez, ex, ey, ez, eax, eay, eaz, probs2)
    return out[0, 0]


# speedup vs baseline: 2.3152x; 1.7007x over previous
import functools

import jax
import jax.numpy as jnp
from jax import lax
from jax.experimental import pallas as pl
from jax.experimental.pallas import tpu as pltpu
from jax.experimental.pallas import tpu_sc as plsc

_K = 20
_KPAD = 32
_NW = 32
_L = 16
_INF = float("inf")


def _rev(x):
    return lax.rev(x, dimensions=(0,))


def _sort_pair(k, v):
    return plsc.sort_key_val(k, v)


def _merge32(lo0, loi0, hi0, hii0, sk, si):
    rk, ri = _rev(sk), _rev(si)
    m = hi0 <= rk
    a = jnp.where(m, hi0, rk)
    ai = jnp.where(m, hii0, ri)
    a, ai = _sort_pair(a, ai)
    rk2, ri2 = _rev(a), _rev(ai)
    m2 = lo0 <= rk2
    lo = jnp.where(m2, lo0, rk2)
    loi = jnp.where(m2, loi0, ri2)
    hi = jnp.where(m2, rk2, lo0)
    hii = jnp.where(m2, ri2, loi0)
    lo, loi = _sort_pair(lo, loi)
    hi, hii = _sort_pair(hi, hii)
    return lo, loi, hi, hii


def _make_sc_knn(F):
    QPW = F // _NW
    NCHUNK = F // _L
    mesh = plsc.VectorSubcoreMesh(core_axis_name="c", subcore_axis_name="s",
                                  num_cores=2, num_subcores=16)

    @functools.partial(
        pl.kernel,
        mesh=mesh,
        compiler_params=pltpu.CompilerParams(needs_layout_passes=False),
        out_type=jax.ShapeDtypeStruct((F, _KPAD), jnp.int32),
        scratch_types=[
            pltpu.VMEM((F + _L,), jnp.float32),
            pltpu.VMEM((F + _L,), jnp.float32),
            pltpu.VMEM((F + _L,), jnp.float32),
            pltpu.VMEM((F,), jnp.float32),
            pltpu.VMEM((F + _L,), jnp.int32),
            pltpu.VMEM((QPW, _KPAD), jnp.int32),
            pltpu.SemaphoreType.DMA,
        ],
    )
    def knn(cx_hbm, cy_hbm, cz_hbm, out_hbm, cx, cy, cz, dbuf, cand_i, outbuf, sem):
        wid = lax.axis_index("s") * 2 + lax.axis_index("c")
        base = wid * QPW
        pltpu.sync_copy(cx_hbm, cx.at[pl.ds(0, F)])
        pltpu.sync_copy(cy_hbm, cy.at[pl.ds(0, F)])
        pltpu.sync_copy(cz_hbm, cz.at[pl.ds(0, F)])
        iota = lax.iota(jnp.int32, _L)

        def per_query(q, carry):
            qxv = cx[pl.ds(base + q, _L)]
            qyv = cy[pl.ds(base + q, _L)]
            qzv = cz[pl.ds(base + q, _L)]
            qx = jnp.full((_L,), qxv[0], jnp.float32)
            qy = jnp.full((_L,), qyv[0], jnp.float32)
            qz = jnp.full((_L,), qzv[0], jnp.float32)

            def p1(i, accs):
                acc_a, acc_b = accs
                o0 = i * (2 * _L)
                dx = cx[pl.ds(o0, _L)] - qx
                dy = cy[pl.ds(o0, _L)] - qy
                dz = cz[pl.ds(o0, _L)] - qz
                d2a = (dx * dx + dy * dy) + dz * dz
                dbuf[pl.ds(o0, _L)] = d2a
                dx = cx[pl.ds(o0 + _L, _L)] - qx
                dy = cy[pl.ds(o0 + _L, _L)] - qy
                dz = cz[pl.ds(o0 + _L, _L)] - qz
                d2b = (dx * dx + dy * dy) + dz * dz
                dbuf[pl.ds(o0 + _L, _L)] = d2b
                return (jnp.minimum(acc_a, d2a), jnp.minimum(acc_b, d2b))

            acc_a, acc_b = lax.fori_loop(
                0, NCHUNK // 2, p1,
                (jnp.full((_L,), _INF), jnp.full((_L,), _INF)))

            s_a, _ = _sort_pair(acc_a, acc_a)
            s_b, _ = _sort_pair(acc_b, acc_b)
            up = jnp.maximum(s_a, _rev(s_b))
            s_u, _ = _sort_pair(up, up)
            thr = jnp.full((_L,), s_u[_K - 1 - _L], jnp.float32)

            def p2(i, cnt):
                v = dbuf[pl.ds(i * _L, _L)]
                m = v <= thr
                idx = iota + i * _L
                plsc.store_compressed(cand_i.at[pl.ds(cnt, _L)], idx, mask=m)
                pc = plsc.all_reduce_population_count(m)
                return cnt + pc[0]

            cnt = lax.fori_loop(0, NCHUNK, p2, jnp.int32(0))

            def p3(j, state):
                lo, loi, hi, hii = state
                iv = cand_i[pl.ds(j * _L, _L)]
                valid = (iota + j * _L) < cnt
                iv = jnp.where(valid, iv, 0)
                vd = plsc.load_gather(dbuf, [iv])
                key = jnp.where(valid, vd, _INF)
                sk, si = _sort_pair(key, iv)
                return _merge32(lo, loi, hi, hii, sk, si)

            nch = (cnt + (_L - 1)) // _L
            init = (jnp.full((_L,), _INF), jnp.zeros((_L,), jnp.int32),
                    jnp.full((_L,), _INF), jnp.zeros((_L,), jnp.int32))
            lo, loi, hi, hii = lax.fori_loop(0, nch, p3, init)

            outbuf[q, pl.ds(0, _L)] = loi
            outbuf[q, pl.ds(_L, _L)] = hii
            return carry

        lax.fori_loop(0, QPW, per_query, jnp.int32(0))
        pltpu.sync_copy(outbuf, out_hbm.at[pl.ds(base, QPW)])

    return knn


def _crossing_body(nex_ref, ney_ref, nez_ref, ex_ref, ey_ref, ez_ref,
                   eax_ref, eay_ref, eaz_ref, probs_ref, out_ref):
    nex = nex_ref[...]
    ney = ney_ref[...]
    nez = nez_ref[...]
    eax = eax_ref[...]
    eay = eay_ref[...]
    eaz = eaz_ref[...]
    acc = jnp.zeros_like(nex)
    for j in range(3):
        e_x = ex_ref[:, j:j + 1]
        e_y = ey_ref[:, j:j + 1]
        e_z = ez_ref[:, j:j + 1]
        cp_x = e_y * nez - e_z * ney
        cp_y = e_z * nex - e_x * nez
        cp_z = e_x * ney - e_y * nex
        denom = (cp_x * e_x + cp_y * e_y) + cp_z * e_z
        tnum = (cp_x * nex + cp_y * ney) + cp_z * nez
        unum = (cp_x * eax + cp_y * eay) + cp_z * eaz
        t = tnum / denom
        u = unum / denom
        mask = (t >= 0.0) & (t <= 1.0) & (u >= 0.0) & (u <= 1.0)
        acc = acc + jnp.where(mask, 1.0, 0.0)
    weighted = acc * probs_ref[...]
    out_ref[...] = jnp.sum(weighted, axis=(0, 1), keepdims=True)


def kernel(vertices, faces, face_probs):
    F = faces.shape[0]
    k = min(_K, F)
    centroids = vertices[faces].mean(axis=1)

    knn = _make_sc_knn(F)
    nearest = knn(centroids[:, 0].copy(), centroids[:, 1].copy(),
                  centroids[:, 2].copy())[:, :k]

    perm = jnp.array([1, 2, 0])
    edges = vertices[faces[:, perm]] - vertices[faces]
    nf = faces[nearest]
    ne = vertices[nf[:, :, perm]] - vertices[nf]

    nex = ne[..., 0].reshape(F, k * 3)
    ney = ne[..., 1].reshape(F, k * 3)
    nez = ne[..., 2].reshape(F, k * 3)
    ex = edges[..., 0]
    ey = edges[..., 1]
    ez = edges[..., 2]
    eax = jnp.tile(ex, (1, k))
    eay = jnp.tile(ey, (1, k))
    eaz = jnp.tile(ez, (1, k))
    probs2 = face_probs[:F, None]

    out = pl.pallas_call(
        _crossing_body,
        out_shape=jax.ShapeDtypeStruct((1, 1), jnp.float32),
    )(nex, ney, nez, ex, ey, ez, eax, eay, eaz, probs2)
    return out[0, 0]
